# Initial kernel scaffold; baseline (speedup 1.0000x reference)
import jax
import jax.numpy as jnp
from jax.experimental import pallas as pl


def kernel(data, K_mem, V_mem, W_g, W_e, b_e):
    raise NotImplementedError("write your pallas kernel here")


# speedup vs baseline: 2.9268x; 2.9268x over previous
import functools

import jax
import jax.numpy as jnp
from jax import lax


def _round_bf16(x):
    u = lax.bitcast_convert_type(x, jnp.uint32)
    lsb = (u >> 16) & jnp.uint32(1)
    u = (u + jnp.uint32(0x7FFF) + lsb) & jnp.uint32(0xFFFF0000)
    return lax.bitcast_convert_type(u, jnp.float32)
from jax.experimental import pallas as pl
from jax.experimental.pallas import tpu as pltpu
from jax.experimental.pallas import tpu_sc as plsc

_T, _D, _E, _M = 2048, 768, 8, 8192
_MEM_TOPK, _GATE_TOPK = 4, 2
_TBLK = 256
_MBLK = 1024
_LANES = 128
_NW = 32
_TPW = _T // _NW
_CH = 16


def _stage1_body(x_ref, k_ref, topi_ref, w_ref, rv_ref, ri_ref):
    m = pl.program_id(0)
    nm = pl.num_programs(0)
    t = pl.program_id(1)
    ts = t * _TBLK
    blk = lax.dot_general(x_ref[...], k_ref[...], (((1,), (1,)), ((), ())),
                          preferred_element_type=jnp.float32)
    col = lax.broadcasted_iota(jnp.int32, (_TBLK, _MBLK), 1) + m * _MBLK
    neg = jnp.float32(-jnp.inf)
    s = blk
    bv, bi = [], []
    for _ in range(_MEM_TOPK):
        mx = jnp.max(s, axis=1, keepdims=True)
        ix = jnp.min(jnp.where(s == mx, col, _M), axis=1, keepdims=True)
        bv.append(mx)
        bi.append(ix)
        s = jnp.where(col == ix, neg, s)
    bv = jnp.concatenate(bv, axis=1)
    bi = jnp.concatenate(bi, axis=1)

    @pl.when(m == 0)
    def _():
        rv_ref[pl.ds(ts, _TBLK), :] = bv
        ri_ref[pl.ds(ts, _TBLK), :] = bi

    @pl.when(m > 0)
    def _():
        cv = jnp.concatenate([rv_ref[pl.ds(ts, _TBLK), :], bv], axis=1)
        ci = jnp.concatenate([ri_ref[pl.ds(ts, _TBLK), :], bi], axis=1)
        pos = lax.broadcasted_iota(jnp.int32, (_TBLK, 2 * _MEM_TOPK), 1)
        nv, ni = [], []
        for _ in range(_MEM_TOPK):
            mx = jnp.max(cv, axis=1, keepdims=True)
            p = jnp.min(jnp.where(cv == mx, pos, 2 * _MEM_TOPK),
                        axis=1, keepdims=True)
            sel = jnp.sum(jnp.where(pos == p, ci, 0), axis=1, keepdims=True)
            nv.append(mx)
            ni.append(sel)
            cv = jnp.where(pos == p, neg, cv)
        rv_ref[pl.ds(ts, _TBLK), :] = jnp.concatenate(nv, axis=1)
        ri_ref[pl.ds(ts, _TBLK), :] = jnp.concatenate(ni, axis=1)

    @pl.when(m == nm - 1)
    def _():
        tv = rv_ref[pl.ds(ts, _TBLK), :]
        e = jnp.exp(tv - tv[:, 0:1])
        w_ref[...] = _round_bf16(e / jnp.sum(e, axis=1, keepdims=True))
        topi_ref[...] = ri_ref[pl.ds(ts, _TBLK), :]


def _stage1(data_bf, k_bf):
    return pl.pallas_call(
        _stage1_body,
        grid=(_M // _MBLK, _T // _TBLK),
        in_specs=[
            pl.BlockSpec((_TBLK, _D), lambda m, t: (t, 0)),
            pl.BlockSpec((_MBLK, _D), lambda m, t: (m, 0)),
        ],
        out_specs=[
            pl.BlockSpec((_TBLK, _MEM_TOPK), lambda m, t: (t, 0)),
            pl.BlockSpec((_TBLK, _MEM_TOPK), lambda m, t: (t, 0)),
        ],
        out_shape=[
            jax.ShapeDtypeStruct((_T, _MEM_TOPK), jnp.int32),
            jax.ShapeDtypeStruct((_T, _MEM_TOPK), jnp.float32),
        ],
        scratch_shapes=[
            pltpu.VMEM((_T, _MEM_TOPK), jnp.float32),
            pltpu.VMEM((_T, _MEM_TOPK), jnp.int32),
        ],
        compiler_params=pltpu.CompilerParams(
            dimension_semantics=("arbitrary", "arbitrary")),
    )(data_bf, k_bf)


def _stage2_body(v_hbm, idx_hbm, wb_hbm, data_hbm, out_hbm,
                 idx_v, wb_v, rows_v, acc_v, sem):
    wid = lax.axis_index("s") * 2 + lax.axis_index("c")
    tok0 = wid * _TPW
    for c in range(_TPW // _CH):
        t0 = tok0 + c * _CH
        pltpu.sync_copy(idx_hbm.at[pl.ds(t0 * _MEM_TOPK, _CH * _MEM_TOPK)],
                        idx_v)
        pltpu.async_copy(v_hbm.at[idx_v], rows_v, sem).wait()
        pltpu.sync_copy(wb_hbm.at[pl.ds(t0 * _MEM_TOPK, _CH * _MEM_TOPK)],
                        wb_v)
        pltpu.sync_copy(data_hbm.at[pl.ds(t0, _CH)], acc_v)
        for t in range(_CH):
            r0 = t * _MEM_TOPK
            ws = [wb_v[r0 + k, :] for k in range(_MEM_TOPK)]

            def _jbody(j, carry, _t=t, _r0=r0, _ws=ws):
                off = j * 16
                mem = _ws[0] * _round_bf16(rows_v[_r0 + 0, pl.ds(off, 16)])
                mem = mem + _ws[1] * _round_bf16(rows_v[_r0 + 1, pl.ds(off, 16)])
                mem = mem + _ws[2] * _round_bf16(rows_v[_r0 + 2, pl.ds(off, 16)])
                mem = mem + _ws[3] * _round_bf16(rows_v[_r0 + 3, pl.ds(off, 16)])
                acc_v[_t, pl.ds(off, 16)] = acc_v[_t, pl.ds(off, 16)] + mem
                return carry

            lax.fori_loop(0, _D // 16, _jbody, 0)
        pltpu.sync_copy(acc_v, out_hbm.at[pl.ds(t0, _CH)])


@functools.cache
def _build_stage2():
    mesh = plsc.VectorSubcoreMesh(core_axis_name="c", subcore_axis_name="s")
    return pl.kernel(
        _stage2_body,
        mesh=mesh,
        out_type=jax.ShapeDtypeStruct((_T, _D), jnp.float32),
        scratch_types=[
            pltpu.VMEM((_CH * _MEM_TOPK,), jnp.int32),
            pltpu.VMEM((_CH * _MEM_TOPK, 16), jnp.float32),
            pltpu.VMEM((_CH * _MEM_TOPK, _D), jnp.float32),
            pltpu.VMEM((_CH, _D), jnp.float32),
            pltpu.SemaphoreType.DMA,
        ],
        compiler_params=pltpu.CompilerParams(needs_layout_passes=False),
    )


def _stage3_body(h_ref, wg_ref, we_ref, be_ref, out_ref):
    h = h_ref[...]
    logits = jnp.dot(h, wg_ref[...], preferred_element_type=jnp.float32)
    col = lax.broadcasted_iota(jnp.int32, (_TBLK, _LANES), 1)
    neg = jnp.float32(-jnp.inf)
    logits = jnp.where(col < _E, logits, neg)
    v0 = jnp.max(logits, axis=1, keepdims=True)
    i0 = jnp.min(jnp.where(logits == v0, col, _E), axis=1, keepdims=True)
    l2 = jnp.where(col == i0, neg, logits)
    v1 = jnp.max(l2, axis=1, keepdims=True)
    i1 = jnp.min(jnp.where(l2 == v1, col, _E), axis=1, keepdims=True)
    e1 = jnp.exp(v1 - v0)
    g0 = 1.0 / (1.0 + e1)
    g1 = e1 / (1.0 + e1)
    gd = jnp.where(col == i0, g0, 0.0) + jnp.where(col == i1, g1, 0.0)
    acc = jnp.zeros((_TBLK, _D), jnp.float32)
    for e in range(_E):
        y = jnp.dot(h, we_ref[e], preferred_element_type=jnp.float32)
        y = y + be_ref[pl.ds(e, 1), :]
        acc = acc + gd[:, e:e + 1] * y
    out_ref[...] = acc


def _stage3(h_bf, wg_bf, we_bf, b_e):
    return pl.pallas_call(
        _stage3_body,
        grid=(_T // _TBLK,),
        in_specs=[
            pl.BlockSpec((_TBLK, _D), lambda t: (t, 0)),
            pl.BlockSpec((_D, _LANES), lambda t: (0, 0)),
            pl.BlockSpec((_E, _D, _D), lambda t: (0, 0, 0)),
            pl.BlockSpec((_E, _D), lambda t: (0, 0)),
        ],
        out_specs=pl.BlockSpec((_TBLK, _D), lambda t: (t, 0)),
        out_shape=jax.ShapeDtypeStruct((_T, _D), jnp.float32),
        compiler_params=pltpu.CompilerParams(
            dimension_semantics=("arbitrary",)),
    )(h_bf, wg_bf, we_bf, b_e)


def kernel(data, K_mem, V_mem, W_g, W_e, b_e):
    bf = jnp.bfloat16
    topi, w = _stage1(data.astype(bf), K_mem.astype(bf))
    w_b = jnp.broadcast_to(w.reshape(-1)[:, None], (_T * _MEM_TOPK, 16))
    h = _build_stage2()(V_mem, topi.reshape(-1), w_b, data)
    wg_bf = jnp.pad(W_g, ((0, 0), (0, _LANES - _E))).astype(bf)
    return _stage3(h.astype(bf), wg_bf, W_e.astype(bf), b_e)


# speedup vs baseline: 2.9959x; 1.0236x over previous
import functools

import jax
import jax.numpy as jnp
from jax import lax
from jax.experimental import pallas as pl
from jax.experimental.pallas import tpu as pltpu
from jax.experimental.pallas import tpu_sc as plsc

_T, _D, _E, _M = 2048, 768, 8, 8192
_MEM_TOPK, _GATE_TOPK = 4, 2
_TBLK = 256
_MBLK = 1024
_LANES = 128
_NW = 32
_CH = 16


def _round_bf16(x):
    u = lax.bitcast_convert_type(x, jnp.uint32)
    lsb = (u >> 16) & jnp.uint32(1)
    u = (u + jnp.uint32(0x7FFF) + lsb) & jnp.uint32(0xFFFF0000)
    return lax.bitcast_convert_type(u, jnp.float32)


def _stage1_body(x_ref, k_ref, topi_ref, w_ref, rv_ref, ri_ref):
    m = pl.program_id(0)
    nm = pl.num_programs(0)
    t = pl.program_id(1)
    ts = t * _TBLK
    blk = lax.dot_general(x_ref[...], k_ref[...], (((1,), (1,)), ((), ())),
                          preferred_element_type=jnp.float32)
    col = lax.broadcasted_iota(jnp.int32, (_TBLK, _MBLK), 1) + m * _MBLK
    neg = jnp.float32(-jnp.inf)
    s = blk
    bv, bi = [], []
    for _ in range(_MEM_TOPK):
        mx = jnp.max(s, axis=1, keepdims=True)
        eq = s == mx
        ix = jnp.min(jnp.where(eq, col, _M), axis=1, keepdims=True)
        bv.append(mx)
        bi.append(ix)
        s = jnp.where(col == ix, neg, s)
    bv = jnp.concatenate(bv, axis=1)
    bi = jnp.concatenate(bi, axis=1)

    @pl.when(m == 0)
    def _():
        rv_ref[pl.ds(ts, _TBLK), :] = bv
        ri_ref[pl.ds(ts, _TBLK), :] = bi

    @pl.when(m > 0)
    def _():
        cv = jnp.concatenate([rv_ref[pl.ds(ts, _TBLK), :], bv], axis=1)
        ci = jnp.concatenate([ri_ref[pl.ds(ts, _TBLK), :], bi], axis=1)
        pos = lax.broadcasted_iota(jnp.int32, (_TBLK, 2 * _MEM_TOPK), 1)
        nv, ni = [], []
        for _ in range(_MEM_TOPK):
            mx = jnp.max(cv, axis=1, keepdims=True)
            p = jnp.min(jnp.where(cv == mx, pos, 2 * _MEM_TOPK),
                        axis=1, keepdims=True)
            sel = jnp.sum(jnp.where(pos == p, ci, 0), axis=1, keepdims=True)
            nv.append(mx)
            ni.append(sel)
            cv = jnp.where(pos == p, neg, cv)
        rv_ref[pl.ds(ts, _TBLK), :] = jnp.concatenate(nv, axis=1)
        ri_ref[pl.ds(ts, _TBLK), :] = jnp.concatenate(ni, axis=1)

    @pl.when(m == nm - 1)
    def _():
        tv = rv_ref[pl.ds(ts, _TBLK), :]
        e = jnp.exp(tv - tv[:, 0:1])
        w_ref[...] = _round_bf16(e / jnp.sum(e, axis=1, keepdims=True))
        topi_ref[...] = ri_ref[pl.ds(ts, _TBLK), :]


def _stage1(data_bf, k_bf):
    th = data_bf.shape[0]
    return pl.pallas_call(
        _stage1_body,
        grid=(_M // _MBLK, th // _TBLK),
        in_specs=[
            pl.BlockSpec((_TBLK, _D), lambda m, t: (t, 0)),
            pl.BlockSpec((_MBLK, _D), lambda m, t: (m, 0)),
        ],
        out_specs=[
            pl.BlockSpec((_TBLK, _MEM_TOPK), lambda m, t: (t, 0)),
            pl.BlockSpec((_TBLK, _MEM_TOPK), lambda m, t: (t, 0)),
        ],
        out_shape=[
            jax.ShapeDtypeStruct((th, _MEM_TOPK), jnp.int32),
            jax.ShapeDtypeStruct((th, _MEM_TOPK), jnp.float32),
        ],
        scratch_shapes=[
            pltpu.VMEM((th, _MEM_TOPK), jnp.float32),
            pltpu.VMEM((th, _MEM_TOPK), jnp.int32),
        ],
        compiler_params=pltpu.CompilerParams(
            dimension_semantics=("arbitrary", "arbitrary")),
    )(data_bf, k_bf)


def _make_stage2_body(tpw):
    def _stage2_body(v_hbm, idx_hbm, wb_hbm, data_hbm, out_hbm,
                     idx_v, wb_v, rows_v, acc_v, sem):
        wid = lax.axis_index("s") * 2 + lax.axis_index("c")
        tok0 = wid * tpw
        for c in range(tpw // _CH):
            t0 = tok0 + c * _CH
            pltpu.sync_copy(
                idx_hbm.at[pl.ds(t0 * _MEM_TOPK, _CH * _MEM_TOPK)], idx_v)
            pltpu.async_copy(v_hbm.at[idx_v], rows_v, sem).wait()
            pltpu.sync_copy(
                wb_hbm.at[pl.ds(t0 * _MEM_TOPK, _CH * _MEM_TOPK)], wb_v)
            pltpu.sync_copy(data_hbm.at[pl.ds(t0, _CH)], acc_v)
            for t in range(_CH):
                r0 = t * _MEM_TOPK
                ws = [wb_v[r0 + k, :] for k in range(_MEM_TOPK)]

                def _jbody(jj, carry, _t=t, _r0=r0, _ws=ws):
                    for u in range(4):
                        off = (jj * 4 + u) * 16
                        mem = _ws[0] * _round_bf16(
                            rows_v[_r0 + 0, pl.ds(off, 16)])
                        mem = mem + _ws[1] * _round_bf16(
                            rows_v[_r0 + 1, pl.ds(off, 16)])
                        mem = mem + _ws[2] * _round_bf16(
                            rows_v[_r0 + 2, pl.ds(off, 16)])
                        mem = mem + _ws[3] * _round_bf16(
                            rows_v[_r0 + 3, pl.ds(off, 16)])
                        acc_v[_t, pl.ds(off, 16)] = (
                            acc_v[_t, pl.ds(off, 16)] + mem)
                    return carry

                lax.fori_loop(0, _D // 64, _jbody, 0)
            pltpu.sync_copy(acc_v, out_hbm.at[pl.ds(t0, _CH)])
    return _stage2_body


@functools.cache
def _build_stage2(th):
    tpw = th // _NW
    mesh = plsc.VectorSubcoreMesh(core_axis_name="c", subcore_axis_name="s")
    return pl.kernel(
        _make_stage2_body(tpw),
        mesh=mesh,
        out_type=jax.ShapeDtypeStruct((th, _D), jnp.float32),
        scratch_types=[
            pltpu.VMEM((_CH * _MEM_TOPK,), jnp.int32),
            pltpu.VMEM((_CH * _MEM_TOPK, 16), jnp.float32),
            pltpu.VMEM((_CH * _MEM_TOPK, _D), jnp.float32),
            pltpu.VMEM((_CH, _D), jnp.float32),
            pltpu.SemaphoreType.DMA,
        ],
        compiler_params=pltpu.CompilerParams(needs_layout_passes=False),
    )


def _stage3_body(h_ref, wg_ref, we_ref, be_ref, out_ref):
    h = h_ref[...]
    logits = jnp.dot(h, wg_ref[...], preferred_element_type=jnp.float32)
    col = lax.broadcasted_iota(jnp.int32, (_TBLK, _LANES), 1)
    neg = jnp.float32(-jnp.inf)
    logits = jnp.where(col < _E, logits, neg)
    v0 = jnp.max(logits, axis=1, keepdims=True)
    i0 = jnp.min(jnp.where(logits == v0, col, _E), axis=1, keepdims=True)
    l2 = jnp.where(col == i0, neg, logits)
    v1 = jnp.max(l2, axis=1, keepdims=True)
    i1 = jnp.min(jnp.where(l2 == v1, col, _E), axis=1, keepdims=True)
    e1 = jnp.exp(v1 - v0)
    g0 = 1.0 / (1.0 + e1)
    g1 = e1 / (1.0 + e1)
    gd = jnp.where(col == i0, g0, 0.0) + jnp.where(col == i1, g1, 0.0)
    acc = jnp.zeros((_TBLK, _D), jnp.float32)
    for e in range(_E):
        y = jnp.dot(h, we_ref[e], preferred_element_type=jnp.float32)
        y = y + be_ref[pl.ds(e, 1), :]
        acc = acc + gd[:, e:e + 1] * y
    out_ref[...] = acc


def _stage3(h_bf, wg_bf, we_bf, b_e):
    th = h_bf.shape[0]
    return pl.pallas_call(
        _stage3_body,
        grid=(th // _TBLK,),
        in_specs=[
            pl.BlockSpec((_TBLK, _D), lambda t: (t, 0)),
            pl.BlockSpec((_D, _LANES), lambda t: (0, 0)),
            pl.BlockSpec((_E, _D, _D), lambda t: (0, 0, 0)),
            pl.BlockSpec((_E, _D), lambda t: (0, 0)),
        ],
        out_specs=pl.BlockSpec((_TBLK, _D), lambda t: (t, 0)),
        out_shape=jax.ShapeDtypeStruct((th, _D), jnp.float32),
        compiler_params=pltpu.CompilerParams(
            dimension_semantics=("arbitrary",)),
    )(h_bf, wg_bf, we_bf, b_e)


def kernel(data, K_mem, V_mem, W_g, W_e, b_e):
    bf = jnp.bfloat16
    k_bf = K_mem.astype(bf)
    wg_bf = jnp.pad(W_g, ((0, 0), (0, _LANES - _E))).astype(bf)
    we_bf = W_e.astype(bf)
    halves = []
    nh = 2
    th = _T // nh
    for i in range(nh):
        dh = lax.slice_in_dim(data, i * th, (i + 1) * th, axis=0)
        topi, w = _stage1(dh.astype(bf), k_bf)
        w_b = jnp.broadcast_to(w.reshape(-1)[:, None], (th * _MEM_TOPK, 16))
        h = _build_stage2(th)(V_mem, topi.reshape(-1), w_b, dh)
        halves.append(_stage3(h.astype(bf), wg_bf, we_bf, b_e))
    return jnp.concatenate(halves, axis=0)


# speedup vs baseline: 3.9894x; 1.3316x over previous
import functools

import jax
import jax.numpy as jnp
from jax import lax
from jax.experimental import pallas as pl
from jax.experimental.pallas import tpu as pltpu
from jax.experimental.pallas import tpu_sc as plsc

_T, _D, _E, _M = 2048, 768, 8, 8192
_MEM_TOPK, _GATE_TOPK = 4, 2
_TBLK = 256
_MBLK = 1024
_LANES = 128
_NW = 32
_CH = 16


def _round_bf16(x):
    u = lax.bitcast_convert_type(x, jnp.uint32)
    lsb = (u >> 16) & jnp.uint32(1)
    u = (u + jnp.uint32(0x7FFF) + lsb) & jnp.uint32(0xFFFF0000)
    return lax.bitcast_convert_type(u, jnp.float32)


def _stage1_body(x_ref, k_ref, topi_ref, w_ref, cv_ref, ci_ref):
    m = pl.program_id(0)
    nm = pl.num_programs(0)
    t = pl.program_id(1)
    ts = t * _TBLK
    blk = lax.dot_general(x_ref[...], k_ref[...], (((1,), (1,)), ((), ())),
                          preferred_element_type=jnp.float32)
    col = lax.broadcasted_iota(jnp.int32, (_TBLK, _MBLK), 1) + m * _MBLK
    neg = jnp.float32(-jnp.inf)
    s = blk
    bv, bi = [], []
    for _ in range(_MEM_TOPK):
        mx = jnp.max(s, axis=1, keepdims=True)
        eq = s == mx
        ix = jnp.minimum(
            jnp.sum(jnp.where(eq, col, 0), axis=1, keepdims=True), _M - 1)
        bv.append(mx)
        bi.append(ix)
        s = jnp.where(col == ix, neg, s)
    cv_ref[m, pl.ds(ts, _TBLK), :] = jnp.concatenate(bv, axis=1)
    ci_ref[m, pl.ds(ts, _TBLK), :] = jnp.concatenate(bi, axis=1)

    @pl.when(m == nm - 1)
    def _():
        nc = nm * _MEM_TOPK
        cv = jnp.concatenate(
            [cv_ref[i, pl.ds(ts, _TBLK), :] for i in range(8)], axis=1)
        ci = jnp.concatenate(
            [ci_ref[i, pl.ds(ts, _TBLK), :] for i in range(8)], axis=1)
        pos = lax.broadcasted_iota(jnp.int32, (_TBLK, nc), 1)
        nv, ni = [], []
        for _ in range(_MEM_TOPK):
            mx = jnp.max(cv, axis=1, keepdims=True)
            p = jnp.min(jnp.where(cv == mx, pos, nc), axis=1, keepdims=True)
            sel = jnp.sum(jnp.where(pos == p, ci, 0), axis=1, keepdims=True)
            nv.append(mx)
            ni.append(sel)
            cv = jnp.where(pos == p, neg, cv)
        tv = jnp.concatenate(nv, axis=1)
        e = jnp.exp(tv - tv[:, 0:1])
        w_ref[...] = _round_bf16(e / jnp.sum(e, axis=1, keepdims=True))
        topi_ref[...] = jnp.concatenate(ni, axis=1)


def _stage1(data_bf, k_bf):
    th = data_bf.shape[0]
    return pl.pallas_call(
        _stage1_body,
        grid=(_M // _MBLK, th // _TBLK),
        in_specs=[
            pl.BlockSpec((_TBLK, _D), lambda m, t: (t, 0)),
            pl.BlockSpec((_MBLK, _D), lambda m, t: (m, 0)),
        ],
        out_specs=[
            pl.BlockSpec((_TBLK, _MEM_TOPK), lambda m, t: (t, 0)),
            pl.BlockSpec((_TBLK, _MEM_TOPK), lambda m, t: (t, 0)),
        ],
        out_shape=[
            jax.ShapeDtypeStruct((th, _MEM_TOPK), jnp.int32),
            jax.ShapeDtypeStruct((th, _MEM_TOPK), jnp.float32),
        ],
        scratch_shapes=[
            pltpu.VMEM((_M // _MBLK, th, _MEM_TOPK), jnp.float32),
            pltpu.VMEM((_M // _MBLK, th, _MEM_TOPK), jnp.int32),
        ],
        compiler_params=pltpu.CompilerParams(
            dimension_semantics=("arbitrary", "arbitrary")),
    )(data_bf, k_bf)


def _make_stage2_body(tpw):
    def _stage2_body(v_hbm, idx_hbm, wb_hbm, data_hbm, out_hbm,
                     idx_v, wb_v, rows_v, acc_v, sem):
        wid = lax.axis_index("s") * 2 + lax.axis_index("c")
        tok0 = wid * tpw
        for c in range(tpw // _CH):
            t0 = tok0 + c * _CH
            pltpu.sync_copy(
                idx_hbm.at[pl.ds(t0 * _MEM_TOPK, _CH * _MEM_TOPK)], idx_v)
            pltpu.async_copy(v_hbm.at[idx_v], rows_v, sem).wait()
            pltpu.sync_copy(
                wb_hbm.at[pl.ds(t0 * _MEM_TOPK, _CH * _MEM_TOPK)], wb_v)
            pltpu.sync_copy(data_hbm.at[pl.ds(t0, _CH)], acc_v)
            for t in range(_CH):
                r0 = t * _MEM_TOPK
                ws = [wb_v[r0 + k, :] for k in range(_MEM_TOPK)]

                def _jbody(jj, carry, _t=t, _r0=r0, _ws=ws):
                    for u in range(4):
                        off = (jj * 4 + u) * 16
                        mem = _ws[0] * _round_bf16(
                            rows_v[_r0 + 0, pl.ds(off, 16)])
                        mem = mem + _ws[1] * _round_bf16(
                            rows_v[_r0 + 1, pl.ds(off, 16)])
                        mem = mem + _ws[2] * _round_bf16(
                            rows_v[_r0 + 2, pl.ds(off, 16)])
                        mem = mem + _ws[3] * _round_bf16(
                            rows_v[_r0 + 3, pl.ds(off, 16)])
                        acc_v[_t, pl.ds(off, 16)] = (
                            acc_v[_t, pl.ds(off, 16)] + mem)
                    return carry

                lax.fori_loop(0, _D // 64, _jbody, 0)
            pltpu.sync_copy(acc_v, out_hbm.at[pl.ds(t0, _CH)])
    return _stage2_body


@functools.cache
def _build_stage2(th):
    tpw = th // _NW
    mesh = plsc.VectorSubcoreMesh(core_axis_name="c", subcore_axis_name="s")
    return pl.kernel(
        _make_stage2_body(tpw),
        mesh=mesh,
        out_type=jax.ShapeDtypeStruct((th, _D), jnp.float32),
        scratch_types=[
            pltpu.VMEM((_CH * _MEM_TOPK,), jnp.int32),
            pltpu.VMEM((_CH * _MEM_TOPK, 16), jnp.float32),
            pltpu.VMEM((_CH * _MEM_TOPK, _D), jnp.float32),
            pltpu.VMEM((_CH, _D), jnp.float32),
            pltpu.SemaphoreType.DMA,
        ],
        compiler_params=pltpu.CompilerParams(needs_layout_passes=False),
    )


def _stage3_body(h_ref, wg_ref, we_ref, be_ref, out_ref):
    h = h_ref[...]
    logits = jnp.dot(h, wg_ref[...], preferred_element_type=jnp.float32)
    col = lax.broadcasted_iota(jnp.int32, (_TBLK, _LANES), 1)
    neg = jnp.float32(-jnp.inf)
    logits = jnp.where(col < _E, logits, neg)
    v0 = jnp.max(logits, axis=1, keepdims=True)
    i0 = jnp.min(jnp.where(logits == v0, col, _E), axis=1, keepdims=True)
    l2 = jnp.where(col == i0, neg, logits)
    v1 = jnp.max(l2, axis=1, keepdims=True)
    i1 = jnp.min(jnp.where(l2 == v1, col, _E), axis=1, keepdims=True)
    e1 = jnp.exp(v1 - v0)
    g0 = 1.0 / (1.0 + e1)
    g1 = e1 / (1.0 + e1)
    gd = jnp.where(col == i0, g0, 0.0) + jnp.where(col == i1, g1, 0.0)
    acc = jnp.zeros((_TBLK, _D), jnp.float32)
    for e in range(_E):
        y = jnp.dot(h, we_ref[e], preferred_element_type=jnp.float32)
        y = y + be_ref[pl.ds(e, 1), :]
        acc = acc + gd[:, e:e + 1] * y
    out_ref[...] = acc


def _stage3(h_bf, wg_bf, we_bf, b_e):
    th = h_bf.shape[0]
    return pl.pallas_call(
        _stage3_body,
        grid=(th // _TBLK,),
        in_specs=[
            pl.BlockSpec((_TBLK, _D), lambda t: (t, 0)),
            pl.BlockSpec((_D, _LANES), lambda t: (0, 0)),
            pl.BlockSpec((_E, _D, _D), lambda t: (0, 0, 0)),
            pl.BlockSpec((_E, _D), lambda t: (0, 0)),
        ],
        out_specs=pl.BlockSpec((_TBLK, _D), lambda t: (t, 0)),
        out_shape=jax.ShapeDtypeStruct((th, _D), jnp.float32),
        compiler_params=pltpu.CompilerParams(
            dimension_semantics=("arbitrary",)),
    )(h_bf, wg_bf, we_bf, b_e)


def kernel(data, K_mem, V_mem, W_g, W_e, b_e):
    bf = jnp.bfloat16
    k_bf = K_mem.astype(bf)
    wg_bf = jnp.pad(W_g, ((0, 0), (0, _LANES - _E))).astype(bf)
    we_bf = W_e.astype(bf)
    halves = []
    nh = 2
    th = _T // nh
    for i in range(nh):
        dh = lax.slice_in_dim(data, i * th, (i + 1) * th, axis=0)
        topi, w = _stage1(dh.astype(bf), k_bf)
        w_b = jnp.broadcast_to(w.reshape(-1)[:, None], (th * _MEM_TOPK, 16))
        h = _build_stage2(th)(V_mem, topi.reshape(-1), w_b, dh)
        halves.append(_stage3(h.astype(bf), wg_bf, we_bf, b_e))
    return jnp.concatenate(halves, axis=0)


# speedup vs baseline: 4.3530x; 1.0911x over previous
import functools

import jax
import jax.numpy as jnp
from jax import lax
from jax.experimental import pallas as pl
from jax.experimental.pallas import tpu as pltpu
from jax.experimental.pallas import tpu_sc as plsc

_T, _D, _E, _M = 2048, 768, 8, 8192
_MEM_TOPK, _GATE_TOPK = 4, 2
_TBLK = 256
_MBLK = 2048
_LANES = 128
_NW = 32
_CH = 16


def _round_bf16(x):
    u = lax.bitcast_convert_type(x, jnp.uint32)
    lsb = (u >> 16) & jnp.uint32(1)
    u = (u + jnp.uint32(0x7FFF) + lsb) & jnp.uint32(0xFFFF0000)
    return lax.bitcast_convert_type(u, jnp.float32)


def _stage1_body(x_ref, k_ref, topi_ref, w_ref, cv_ref, ci_ref):
    m = pl.program_id(0)
    nm = pl.num_programs(0)
    t = pl.program_id(1)
    ts = t * _TBLK
    blk = lax.dot_general(x_ref[...], k_ref[...], (((1,), (1,)), ((), ())),
                          preferred_element_type=jnp.float32)
    col = lax.broadcasted_iota(jnp.int32, (_TBLK, _MBLK), 1) + m * _MBLK
    neg = jnp.float32(-jnp.inf)
    s = blk
    bv, bi = [], []
    for _ in range(_MEM_TOPK):
        mx = jnp.max(s, axis=1, keepdims=True)
        eq = s == mx
        ix = jnp.minimum(
            jnp.sum(jnp.where(eq, col, 0), axis=1, keepdims=True), _M - 1)
        bv.append(mx)
        bi.append(ix)
        s = jnp.where(col == ix, neg, s)
    cv_ref[m, pl.ds(ts, _TBLK), :] = jnp.concatenate(bv, axis=1)
    ci_ref[m, pl.ds(ts, _TBLK), :] = jnp.concatenate(bi, axis=1)

    @pl.when(m == nm - 1)
    def _():
        nc = nm * _MEM_TOPK
        cv = jnp.concatenate(
            [cv_ref[i, pl.ds(ts, _TBLK), :] for i in range(_M // _MBLK)],
            axis=1)
        ci = jnp.concatenate(
            [ci_ref[i, pl.ds(ts, _TBLK), :] for i in range(_M // _MBLK)],
            axis=1)
        pos = lax.broadcasted_iota(jnp.int32, (_TBLK, nc), 1)
        nv, ni = [], []
        for _ in range(_MEM_TOPK):
            mx = jnp.max(cv, axis=1, keepdims=True)
            p = jnp.min(jnp.where(cv == mx, pos, nc), axis=1, keepdims=True)
            sel = jnp.sum(jnp.where(pos == p, ci, 0), axis=1, keepdims=True)
            nv.append(mx)
            ni.append(sel)
            cv = jnp.where(pos == p, neg, cv)
        tv = jnp.concatenate(nv, axis=1)
        e = jnp.exp(tv - tv[:, 0:1])
        w_ref[...] = _round_bf16(e / jnp.sum(e, axis=1, keepdims=True))
        topi_ref[...] = jnp.concatenate(ni, axis=1)


def _stage1(data_bf, k_bf):
    th = data_bf.shape[0]
    return pl.pallas_call(
        _stage1_body,
        grid=(_M // _MBLK, th // _TBLK),
        in_specs=[
            pl.BlockSpec((_TBLK, _D), lambda m, t: (t, 0)),
            pl.BlockSpec((_MBLK, _D), lambda m, t: (m, 0)),
        ],
        out_specs=[
            pl.BlockSpec((_TBLK, _MEM_TOPK), lambda m, t: (t, 0)),
            pl.BlockSpec((_TBLK, _MEM_TOPK), lambda m, t: (t, 0)),
        ],
        out_shape=[
            jax.ShapeDtypeStruct((th, _MEM_TOPK), jnp.int32),
            jax.ShapeDtypeStruct((th, _MEM_TOPK), jnp.float32),
        ],
        scratch_shapes=[
            pltpu.VMEM((_M // _MBLK, th, _MEM_TOPK), jnp.float32),
            pltpu.VMEM((_M // _MBLK, th, _MEM_TOPK), jnp.int32),
        ],
        compiler_params=pltpu.CompilerParams(
            dimension_semantics=("arbitrary", "arbitrary")),
    )(data_bf, k_bf)


def _make_stage2_body(tpw):
    def _stage2_body(v_hbm, idx_hbm, wb_hbm, data_hbm, out_hbm,
                     idx_v, wb_v, rows_v, acc_v, sem):
        wid = lax.axis_index("s") * 2 + lax.axis_index("c")
        tok0 = wid * tpw
        for c in range(tpw // _CH):
            t0 = tok0 + c * _CH
            pltpu.sync_copy(
                idx_hbm.at[pl.ds(t0 * _MEM_TOPK, _CH * _MEM_TOPK)], idx_v)
            pltpu.async_copy(v_hbm.at[idx_v], rows_v, sem).wait()
            pltpu.sync_copy(
                wb_hbm.at[pl.ds(t0 * _MEM_TOPK, _CH * _MEM_TOPK)], wb_v)
            pltpu.sync_copy(data_hbm.at[pl.ds(t0, _CH)], acc_v)
            for t in range(_CH):
                r0 = t * _MEM_TOPK
                ws = [wb_v[r0 + k, :] for k in range(_MEM_TOPK)]

                def _jbody(jj, carry, _t=t, _r0=r0, _ws=ws):
                    for u in range(4):
                        off = (jj * 4 + u) * 16
                        mem = _ws[0] * _round_bf16(
                            rows_v[_r0 + 0, pl.ds(off, 16)])
                        mem = mem + _ws[1] * _round_bf16(
                            rows_v[_r0 + 1, pl.ds(off, 16)])
                        mem = mem + _ws[2] * _round_bf16(
                            rows_v[_r0 + 2, pl.ds(off, 16)])
                        mem = mem + _ws[3] * _round_bf16(
                            rows_v[_r0 + 3, pl.ds(off, 16)])
                        acc_v[_t, pl.ds(off, 16)] = (
                            acc_v[_t, pl.ds(off, 16)] + mem)
                    return carry

                lax.fori_loop(0, _D // 64, _jbody, 0)
            pltpu.sync_copy(acc_v, out_hbm.at[pl.ds(t0, _CH)])
    return _stage2_body


@functools.cache
def _build_stage2(th):
    tpw = th // _NW
    mesh = plsc.VectorSubcoreMesh(core_axis_name="c", subcore_axis_name="s")
    return pl.kernel(
        _make_stage2_body(tpw),
        mesh=mesh,
        out_type=jax.ShapeDtypeStruct((th, _D), jnp.float32),
        scratch_types=[
            pltpu.VMEM((_CH * _MEM_TOPK,), jnp.int32),
            pltpu.VMEM((_CH * _MEM_TOPK, 16), jnp.float32),
            pltpu.VMEM((_CH * _MEM_TOPK, _D), jnp.float32),
            pltpu.VMEM((_CH, _D), jnp.float32),
            pltpu.SemaphoreType.DMA,
        ],
        compiler_params=pltpu.CompilerParams(needs_layout_passes=False),
    )


def _stage3_body(h_ref, wg_ref, we_ref, be_ref, out_ref):
    h = h_ref[...]
    logits = jnp.dot(h, wg_ref[...], preferred_element_type=jnp.float32)
    col = lax.broadcasted_iota(jnp.int32, (_TBLK, _LANES), 1)
    neg = jnp.float32(-jnp.inf)
    logits = jnp.where(col < _E, logits, neg)
    v0 = jnp.max(logits, axis=1, keepdims=True)
    i0 = jnp.min(jnp.where(logits == v0, col, _E), axis=1, keepdims=True)
    l2 = jnp.where(col == i0, neg, logits)
    v1 = jnp.max(l2, axis=1, keepdims=True)
    i1 = jnp.min(jnp.where(l2 == v1, col, _E), axis=1, keepdims=True)
    e1 = jnp.exp(v1 - v0)
    g0 = 1.0 / (1.0 + e1)
    g1 = e1 / (1.0 + e1)
    gd = jnp.where(col == i0, g0, 0.0) + jnp.where(col == i1, g1, 0.0)
    acc = jnp.zeros((_TBLK, _D), jnp.float32)
    for e in range(_E):
        y = jnp.dot(h, we_ref[e], preferred_element_type=jnp.float32)
        y = y + be_ref[pl.ds(e, 1), :]
        acc = acc + gd[:, e:e + 1] * y
    out_ref[...] = acc


def _stage3(h_bf, wg_bf, we_bf, b_e):
    th = h_bf.shape[0]
    return pl.pallas_call(
        _stage3_body,
        grid=(th // _TBLK,),
        in_specs=[
            pl.BlockSpec((_TBLK, _D), lambda t: (t, 0)),
            pl.BlockSpec((_D, _LANES), lambda t: (0, 0)),
            pl.BlockSpec((_E, _D, _D), lambda t: (0, 0, 0)),
            pl.BlockSpec((_E, _D), lambda t: (0, 0)),
        ],
        out_specs=pl.BlockSpec((_TBLK, _D), lambda t: (t, 0)),
        out_shape=jax.ShapeDtypeStruct((th, _D), jnp.float32),
        compiler_params=pltpu.CompilerParams(
            dimension_semantics=("arbitrary",)),
    )(h_bf, wg_bf, we_bf, b_e)


def kernel(data, K_mem, V_mem, W_g, W_e, b_e):
    bf = jnp.bfloat16
    k_bf = K_mem.astype(bf)
    wg_bf = jnp.pad(W_g, ((0, 0), (0, _LANES - _E))).astype(bf)
    we_bf = W_e.astype(bf)
    halves = []
    nh = 2
    th = _T // nh
    for i in range(nh):
        dh = lax.slice_in_dim(data, i * th, (i + 1) * th, axis=0)
        topi, w = _stage1(dh.astype(bf), k_bf)
        w_b = jnp.broadcast_to(w.reshape(-1)[:, None], (th * _MEM_TOPK, 16))
        h = _build_stage2(th)(V_mem, topi.reshape(-1), w_b, dh)
        halves.append(_stage3(h.astype(bf), wg_bf, we_bf, b_e))
    return jnp.concatenate(halves, axis=0)


# speedup vs baseline: 4.5351x; 1.0418x over previous
import functools

import jax
import jax.numpy as jnp
from jax import lax
from jax.experimental import pallas as pl
from jax.experimental.pallas import tpu as pltpu
from jax.experimental.pallas import tpu_sc as plsc

_T, _D, _E, _M = 2048, 768, 8, 8192
_MEM_TOPK, _GATE_TOPK = 4, 2
_TBLK = 256
_MBLK = 4096
_LANES = 128
_NW = 32
_CH = 16


def _round_bf16(x):
    u = lax.bitcast_convert_type(x, jnp.uint32)
    lsb = (u >> 16) & jnp.uint32(1)
    u = (u + jnp.uint32(0x7FFF) + lsb) & jnp.uint32(0xFFFF0000)
    return lax.bitcast_convert_type(u, jnp.float32)


def _stage1_body(x_ref, k_ref, topi_ref, w_ref, cv_ref, ci_ref):
    m = pl.program_id(0)
    nm = pl.num_programs(0)
    t = pl.program_id(1)
    ts = t * _TBLK
    blk = lax.dot_general(x_ref[...], k_ref[...], (((1,), (1,)), ((), ())),
                          preferred_element_type=jnp.float32)
    col = lax.broadcasted_iota(jnp.int32, (_TBLK, _MBLK), 1) + m * _MBLK
    neg = jnp.float32(-jnp.inf)
    s = blk
    bv, bi = [], []
    for _ in range(_MEM_TOPK):
        mx = jnp.max(s, axis=1, keepdims=True)
        eq = s == mx
        ix = jnp.minimum(
            jnp.sum(jnp.where(eq, col, 0), axis=1, keepdims=True), _M - 1)
        bv.append(mx)
        bi.append(ix)
        s = jnp.where(col == ix, neg, s)
    cv_ref[m, pl.ds(ts, _TBLK), :] = jnp.concatenate(bv, axis=1)
    ci_ref[m, pl.ds(ts, _TBLK), :] = jnp.concatenate(bi, axis=1)

    @pl.when(m == nm - 1)
    def _():
        nc = nm * _MEM_TOPK
        cv = jnp.concatenate(
            [cv_ref[i, pl.ds(ts, _TBLK), :] for i in range(_M // _MBLK)],
            axis=1)
        ci = jnp.concatenate(
            [ci_ref[i, pl.ds(ts, _TBLK), :] for i in range(_M // _MBLK)],
            axis=1)
        pos = lax.broadcasted_iota(jnp.int32, (_TBLK, nc), 1)
        nv, ni = [], []
        for _ in range(_MEM_TOPK):
            mx = jnp.max(cv, axis=1, keepdims=True)
            p = jnp.min(jnp.where(cv == mx, pos, nc), axis=1, keepdims=True)
            sel = jnp.sum(jnp.where(pos == p, ci, 0), axis=1, keepdims=True)
            nv.append(mx)
            ni.append(sel)
            cv = jnp.where(pos == p, neg, cv)
        tv = jnp.concatenate(nv, axis=1)
        e = jnp.exp(tv - tv[:, 0:1])
        w_ref[...] = _round_bf16(e / jnp.sum(e, axis=1, keepdims=True))
        topi_ref[...] = jnp.concatenate(ni, axis=1)


def _stage1(data_bf, k_bf):
    th = data_bf.shape[0]
    return pl.pallas_call(
        _stage1_body,
        grid=(_M // _MBLK, th // _TBLK),
        in_specs=[
            pl.BlockSpec((_TBLK, _D), lambda m, t: (t, 0)),
            pl.BlockSpec((_MBLK, _D), lambda m, t: (m, 0)),
        ],
        out_specs=[
            pl.BlockSpec((_TBLK, _MEM_TOPK), lambda m, t: (t, 0)),
            pl.BlockSpec((_TBLK, _MEM_TOPK), lambda m, t: (t, 0)),
        ],
        out_shape=[
            jax.ShapeDtypeStruct((th, _MEM_TOPK), jnp.int32),
            jax.ShapeDtypeStruct((th, _MEM_TOPK), jnp.float32),
        ],
        scratch_shapes=[
            pltpu.VMEM((_M // _MBLK, th, _MEM_TOPK), jnp.float32),
            pltpu.VMEM((_M // _MBLK, th, _MEM_TOPK), jnp.int32),
        ],
        compiler_params=pltpu.CompilerParams(
            dimension_semantics=("arbitrary", "arbitrary")),
    )(data_bf, k_bf)


def _make_stage2_body(tpw):
    def _stage2_body(v_hbm, idx_hbm, wb_hbm, data_hbm, out_hbm,
                     idx_v, wb_v, rows_v, acc_v, sem):
        wid = lax.axis_index("s") * 2 + lax.axis_index("c")
        tok0 = wid * tpw
        for c in range(tpw // _CH):
            t0 = tok0 + c * _CH
            pltpu.sync_copy(
                idx_hbm.at[pl.ds(t0 * _MEM_TOPK, _CH * _MEM_TOPK)], idx_v)
            pltpu.async_copy(v_hbm.at[idx_v], rows_v, sem).wait()
            pltpu.sync_copy(
                wb_hbm.at[pl.ds(t0 * _MEM_TOPK, _CH * _MEM_TOPK)], wb_v)
            pltpu.sync_copy(data_hbm.at[pl.ds(t0, _CH)], acc_v)
            for t in range(_CH):
                r0 = t * _MEM_TOPK
                ws = [wb_v[r0 + k, :] for k in range(_MEM_TOPK)]

                def _jbody(jj, carry, _t=t, _r0=r0, _ws=ws):
                    for u in range(4):
                        off = (jj * 4 + u) * 16
                        mem = _ws[0] * _round_bf16(
                            rows_v[_r0 + 0, pl.ds(off, 16)])
                        mem = mem + _ws[1] * _round_bf16(
                            rows_v[_r0 + 1, pl.ds(off, 16)])
                        mem = mem + _ws[2] * _round_bf16(
                            rows_v[_r0 + 2, pl.ds(off, 16)])
                        mem = mem + _ws[3] * _round_bf16(
                            rows_v[_r0 + 3, pl.ds(off, 16)])
                        acc_v[_t, pl.ds(off, 16)] = (
                            acc_v[_t, pl.ds(off, 16)] + mem)
                    return carry

                lax.fori_loop(0, _D // 64, _jbody, 0)
            pltpu.sync_copy(acc_v, out_hbm.at[pl.ds(t0, _CH)])
    return _stage2_body


@functools.cache
def _build_stage2(th):
    tpw = th // _NW
    mesh = plsc.VectorSubcoreMesh(core_axis_name="c", subcore_axis_name="s")
    return pl.kernel(
        _make_stage2_body(tpw),
        mesh=mesh,
        out_type=jax.ShapeDtypeStruct((th, _D), jnp.float32),
        scratch_types=[
            pltpu.VMEM((_CH * _MEM_TOPK,), jnp.int32),
            pltpu.VMEM((_CH * _MEM_TOPK, 16), jnp.float32),
            pltpu.VMEM((_CH * _MEM_TOPK, _D), jnp.float32),
            pltpu.VMEM((_CH, _D), jnp.float32),
            pltpu.SemaphoreType.DMA,
        ],
        compiler_params=pltpu.CompilerParams(needs_layout_passes=False),
    )


def _stage3_body(h_ref, wg_ref, we_ref, be_ref, out_ref):
    h = h_ref[...]
    logits = jnp.dot(h, wg_ref[...], preferred_element_type=jnp.float32)
    col = lax.broadcasted_iota(jnp.int32, (_TBLK, _LANES), 1)
    neg = jnp.float32(-jnp.inf)
    logits = jnp.where(col < _E, logits, neg)
    v0 = jnp.max(logits, axis=1, keepdims=True)
    i0 = jnp.min(jnp.where(logits == v0, col, _E), axis=1, keepdims=True)
    l2 = jnp.where(col == i0, neg, logits)
    v1 = jnp.max(l2, axis=1, keepdims=True)
    i1 = jnp.min(jnp.where(l2 == v1, col, _E), axis=1, keepdims=True)
    e1 = jnp.exp(v1 - v0)
    g0 = 1.0 / (1.0 + e1)
    g1 = e1 / (1.0 + e1)
    gd = jnp.where(col == i0, g0, 0.0) + jnp.where(col == i1, g1, 0.0)
    acc = jnp.zeros((_TBLK, _D), jnp.float32)
    for e in range(_E):
        y = jnp.dot(h, we_ref[e], preferred_element_type=jnp.float32)
        y = y + be_ref[pl.ds(e, 1), :]
        acc = acc + gd[:, e:e + 1] * y
    out_ref[...] = acc


def _stage3(h_bf, wg_bf, we_bf, b_e):
    th = h_bf.shape[0]
    return pl.pallas_call(
        _stage3_body,
        grid=(th // _TBLK,),
        in_specs=[
            pl.BlockSpec((_TBLK, _D), lambda t: (t, 0)),
            pl.BlockSpec((_D, _LANES), lambda t: (0, 0)),
            pl.BlockSpec((_E, _D, _D), lambda t: (0, 0, 0)),
            pl.BlockSpec((_E, _D), lambda t: (0, 0)),
        ],
        out_specs=pl.BlockSpec((_TBLK, _D), lambda t: (t, 0)),
        out_shape=jax.ShapeDtypeStruct((th, _D), jnp.float32),
        compiler_params=pltpu.CompilerParams(
            dimension_semantics=("arbitrary",)),
    )(h_bf, wg_bf, we_bf, b_e)


def kernel(data, K_mem, V_mem, W_g, W_e, b_e):
    bf = jnp.bfloat16
    k_bf = K_mem.astype(bf)
    wg_bf = jnp.pad(W_g, ((0, 0), (0, _LANES - _E))).astype(bf)
    we_bf = W_e.astype(bf)
    halves = []
    nh = 2
    th = _T // nh
    for i in range(nh):
        dh = lax.slice_in_dim(data, i * th, (i + 1) * th, axis=0)
        topi, w = _stage1(dh.astype(bf), k_bf)
        w_b = jnp.broadcast_to(w.reshape(-1)[:, None], (th * _MEM_TOPK, 16))
        h = _build_stage2(th)(V_mem, topi.reshape(-1), w_b, dh)
        halves.append(_stage3(h.astype(bf), wg_bf, we_bf, b_e))
    return jnp.concatenate(halves, axis=0)


# speedup vs baseline: 4.5400x; 1.0011x over previous
import functools

import jax
import jax.numpy as jnp
from jax import lax
from jax.experimental import pallas as pl
from jax.experimental.pallas import tpu as pltpu
from jax.experimental.pallas import tpu_sc as plsc

_T, _D, _E, _M = 2048, 768, 8, 8192
_MEM_TOPK, _GATE_TOPK = 4, 2
_TBLK = 256
_MBLK = 4096
_LANES = 128
_NW = 32
_CH = 16


def _round_bf16(x):
    u = lax.bitcast_convert_type(x, jnp.uint32)
    lsb = (u >> 16) & jnp.uint32(1)
    u = (u + jnp.uint32(0x7FFF) + lsb) & jnp.uint32(0xFFFF0000)
    return lax.bitcast_convert_type(u, jnp.float32)


def _stage1_body(x_ref, k_ref, topi_ref, w_ref, cv_ref, ci_ref):
    m = pl.program_id(0)
    nm = pl.num_programs(0)
    t = pl.program_id(1)
    ts = t * _TBLK
    blk = lax.dot_general(x_ref[...], k_ref[...], (((1,), (1,)), ((), ())),
                          preferred_element_type=jnp.float32)
    col = lax.broadcasted_iota(jnp.int32, (_TBLK, _MBLK), 1)
    neg = jnp.float32(-jnp.inf)
    s = blk
    bv, bi = [], []
    for k in range(_MEM_TOPK):
        mx = jnp.max(s, axis=1, keepdims=True)
        eq = s == mx
        ix = jnp.minimum(
            jnp.sum(jnp.where(eq, col, 0), axis=1, keepdims=True), _MBLK - 1)
        bv.append(mx)
        bi.append(ix)
        if k < _MEM_TOPK - 1:
            s = jnp.where(col == ix, neg, s)
    cv_ref[m, pl.ds(ts, _TBLK), :] = jnp.concatenate(bv, axis=1)
    ci_ref[m, pl.ds(ts, _TBLK), :] = jnp.concatenate(bi, axis=1) + m * _MBLK

    @pl.when(m == nm - 1)
    def _():
        nc = nm * _MEM_TOPK
        cv = jnp.concatenate(
            [cv_ref[i, pl.ds(ts, _TBLK), :] for i in range(_M // _MBLK)],
            axis=1)
        ci = jnp.concatenate(
            [ci_ref[i, pl.ds(ts, _TBLK), :] for i in range(_M // _MBLK)],
            axis=1)
        pos = lax.broadcasted_iota(jnp.int32, (_TBLK, nc), 1)
        nv, ni = [], []
        for _ in range(_MEM_TOPK):
            mx = jnp.max(cv, axis=1, keepdims=True)
            p = jnp.min(jnp.where(cv == mx, pos, nc), axis=1, keepdims=True)
            sel = jnp.sum(jnp.where(pos == p, ci, 0), axis=1, keepdims=True)
            nv.append(mx)
            ni.append(sel)
            cv = jnp.where(pos == p, neg, cv)
        tv = jnp.concatenate(nv, axis=1)
        e = jnp.exp(tv - tv[:, 0:1])
        w_ref[...] = _round_bf16(e / jnp.sum(e, axis=1, keepdims=True))
        topi_ref[...] = jnp.concatenate(ni, axis=1)


def _stage1(data_bf, k_bf):
    th = data_bf.shape[0]
    return pl.pallas_call(
        _stage1_body,
        grid=(_M // _MBLK, th // _TBLK),
        in_specs=[
            pl.BlockSpec((_TBLK, _D), lambda m, t: (t, 0)),
            pl.BlockSpec((_MBLK, _D), lambda m, t: (m, 0)),
        ],
        out_specs=[
            pl.BlockSpec((_TBLK, _MEM_TOPK), lambda m, t: (t, 0)),
            pl.BlockSpec((_TBLK, _MEM_TOPK), lambda m, t: (t, 0)),
        ],
        out_shape=[
            jax.ShapeDtypeStruct((th, _MEM_TOPK), jnp.int32),
            jax.ShapeDtypeStruct((th, _MEM_TOPK), jnp.float32),
        ],
        scratch_shapes=[
            pltpu.VMEM((_M // _MBLK, th, _MEM_TOPK), jnp.float32),
            pltpu.VMEM((_M // _MBLK, th, _MEM_TOPK), jnp.int32),
        ],
        compiler_params=pltpu.CompilerParams(
            dimension_semantics=("arbitrary", "arbitrary")),
    )(data_bf, k_bf)


def _make_stage2_body(tpw):
    def _stage2_body(v_hbm, idx_hbm, wb_hbm, data_hbm, out_hbm,
                     idx_v, wb_v, rows_v, acc_v, sem):
        wid = lax.axis_index("s") * 2 + lax.axis_index("c")
        tok0 = wid * tpw
        for c in range(tpw // _CH):
            t0 = tok0 + c * _CH
            pltpu.sync_copy(
                idx_hbm.at[pl.ds(t0 * _MEM_TOPK, _CH * _MEM_TOPK)], idx_v)
            pltpu.async_copy(v_hbm.at[idx_v], rows_v, sem).wait()
            pltpu.sync_copy(
                wb_hbm.at[pl.ds(t0 * _MEM_TOPK, _CH * _MEM_TOPK)], wb_v)
            pltpu.sync_copy(data_hbm.at[pl.ds(t0, _CH)], acc_v)
            for t in range(_CH):
                r0 = t * _MEM_TOPK
                ws = [wb_v[r0 + k, :] for k in range(_MEM_TOPK)]

                def _jbody(jj, carry, _t=t, _r0=r0, _ws=ws):
                    for u in range(4):
                        off = (jj * 4 + u) * 16
                        mem = _ws[0] * _round_bf16(
                            rows_v[_r0 + 0, pl.ds(off, 16)])
                        mem = mem + _ws[1] * _round_bf16(
                            rows_v[_r0 + 1, pl.ds(off, 16)])
                        mem = mem + _ws[2] * _round_bf16(
                            rows_v[_r0 + 2, pl.ds(off, 16)])
                        mem = mem + _ws[3] * _round_bf16(
                            rows_v[_r0 + 3, pl.ds(off, 16)])
                        acc_v[_t, pl.ds(off, 16)] = (
                            acc_v[_t, pl.ds(off, 16)] + mem)
                    return carry

                lax.fori_loop(0, _D // 64, _jbody, 0)
            pltpu.sync_copy(acc_v, out_hbm.at[pl.ds(t0, _CH)])
    return _stage2_body


@functools.cache
def _build_stage2(th):
    tpw = th // _NW
    mesh = plsc.VectorSubcoreMesh(core_axis_name="c", subcore_axis_name="s")
    return pl.kernel(
        _make_stage2_body(tpw),
        mesh=mesh,
        out_type=jax.ShapeDtypeStruct((th, _D), jnp.float32),
        scratch_types=[
            pltpu.VMEM((_CH * _MEM_TOPK,), jnp.int32),
            pltpu.VMEM((_CH * _MEM_TOPK, 16), jnp.float32),
            pltpu.VMEM((_CH * _MEM_TOPK, _D), jnp.float32),
            pltpu.VMEM((_CH, _D), jnp.float32),
            pltpu.SemaphoreType.DMA,
        ],
        compiler_params=pltpu.CompilerParams(needs_layout_passes=False),
    )


def _stage3_body(h_ref, wg_ref, we_ref, be_ref, out_ref):
    h = h_ref[...]
    logits = jnp.dot(h, wg_ref[...], preferred_element_type=jnp.float32)
    col = lax.broadcasted_iota(jnp.int32, (_TBLK, _LANES), 1)
    neg = jnp.float32(-jnp.inf)
    logits = jnp.where(col < _E, logits, neg)
    v0 = jnp.max(logits, axis=1, keepdims=True)
    i0 = jnp.min(jnp.where(logits == v0, col, _E), axis=1, keepdims=True)
    l2 = jnp.where(col == i0, neg, logits)
    v1 = jnp.max(l2, axis=1, keepdims=True)
    i1 = jnp.min(jnp.where(l2 == v1, col, _E), axis=1, keepdims=True)
    e1 = jnp.exp(v1 - v0)
    g0 = 1.0 / (1.0 + e1)
    g1 = e1 / (1.0 + e1)
    gd = jnp.where(col == i0, g0, 0.0) + jnp.where(col == i1, g1, 0.0)
    acc = jnp.zeros((_TBLK, _D), jnp.float32)
    for e in range(_E):
        y = jnp.dot(h, we_ref[e], preferred_element_type=jnp.float32)
        y = y + be_ref[pl.ds(e, 1), :]
        acc = acc + gd[:, e:e + 1] * y
    out_ref[...] = acc


def _stage3(h_bf, wg_bf, we_bf, b_e):
    th = h_bf.shape[0]
    return pl.pallas_call(
        _stage3_body,
        grid=(th // _TBLK,),
        in_specs=[
            pl.BlockSpec((_TBLK, _D), lambda t: (t, 0)),
            pl.BlockSpec((_D, _LANES), lambda t: (0, 0)),
            pl.BlockSpec((_E, _D, _D), lambda t: (0, 0, 0)),
            pl.BlockSpec((_E, _D), lambda t: (0, 0)),
        ],
        out_specs=pl.BlockSpec((_TBLK, _D), lambda t: (t, 0)),
        out_shape=jax.ShapeDtypeStruct((th, _D), jnp.float32),
        compiler_params=pltpu.CompilerParams(
            dimension_semantics=("arbitrary",)),
    )(h_bf, wg_bf, we_bf, b_e)


def kernel(data, K_mem, V_mem, W_g, W_e, b_e):
    bf = jnp.bfloat16
    k_bf = K_mem.astype(bf)
    wg_bf = jnp.pad(W_g, ((0, 0), (0, _LANES - _E))).astype(bf)
    we_bf = W_e.astype(bf)
    halves = []
    nh = 2
    th = _T // nh
    for i in range(nh):
        dh = lax.slice_in_dim(data, i * th, (i + 1) * th, axis=0)
        topi, w = _stage1(dh.astype(bf), k_bf)
        w_b = jnp.broadcast_to(w.reshape(-1)[:, None], (th * _MEM_TOPK, 16))
        h = _build_stage2(th)(V_mem, topi.reshape(-1), w_b, dh)
        halves.append(_stage3(h.astype(bf), wg_bf, we_bf, b_e))
    return jnp.concatenate(halves, axis=0)
